# baseline (speedup 1.0000x reference)
import math

import jax
import jax.numpy as jnp
from jax.experimental import pallas as pl
from jax.experimental.pallas import tpu as pltpu

_N = 4096
_F = 128
_H = 64
_NL = 8
_BR = 2048
_NB = _N // _BR
_ALPHA = 0.1


def _gcn_body(adj_ref, gra_ref, wg0_ref, bg0_ref, wce_ref, wg1_ref, bg1_ref,
              out_ref, xf0, xf1, xb0, xb1, h0s):
    i = pl.program_id(0)
    r = pl.program_id(1)
    row0 = r * _BR

    @pl.when(jnp.logical_and(i == 0, r == 0))
    def _prologue():
        x0 = jnp.maximum(
            jnp.dot(gra_ref[...], wg0_ref[...],
                    preferred_element_type=jnp.float32) + bg0_ref[...], 0.0)
        xf0[...] = x0
        xb0[...] = ((1.0 - _ALPHA) * x0).astype(jnp.bfloat16)
        h0s[...] = _ALPHA * x0

    def step(src_f, src_b, dst_f, dst_b):
        hi = jnp.dot(adj_ref[...], src_b[...],
                     preferred_element_type=jnp.float32)
        support = hi + h0s[pl.ds(row0, _BR), :]
        out = jnp.dot(support, wce_ref[0],
                      preferred_element_type=jnp.float32)
        xn = jnp.maximum(out + src_f[pl.ds(row0, _BR), :], 0.0)
        dst_f[pl.ds(row0, _BR), :] = xn
        dst_b[pl.ds(row0, _BR), :] = ((1.0 - _ALPHA) * xn).astype(jnp.bfloat16)

        @pl.when(i == _NL - 1)
        def _epilogue():
            out_ref[...] = (jnp.dot(xn, wg1_ref[...],
                                    preferred_element_type=jnp.float32)
                            + bg1_ref[...])

    @pl.when(i % 2 == 0)
    def _even():
        step(xf0, xb0, xf1, xb1)

    @pl.when(i % 2 == 1)
    def _odd():
        step(xf1, xb1, xf0, xb0)


def _graph_branch(adj_bf, pro_gra, Wg0, bg0, wce, wg1p, bg1p):
    return pl.pallas_call(
        _gcn_body,
        grid=(_NL, _NB),
        in_specs=[
            pl.BlockSpec((_BR, _N), lambda i, r: (r, 0)),
            pl.BlockSpec((_N, _F), lambda i, r: (0, 0)),
            pl.BlockSpec((_F, _H), lambda i, r: (0, 0)),
            pl.BlockSpec((1, _H), lambda i, r: (0, 0)),
            pl.BlockSpec((1, _H, _H), lambda i, r: (i, 0, 0)),
            pl.BlockSpec((_H, 128), lambda i, r: (0, 0)),
            pl.BlockSpec((1, 128), lambda i, r: (0, 0)),
        ],
        out_specs=pl.BlockSpec((_BR, 128), lambda i, r: (r, 0)),
        out_shape=jax.ShapeDtypeStruct((_N, 128), jnp.float32),
        scratch_shapes=[
            pltpu.VMEM((_N, _H), jnp.float32),
            pltpu.VMEM((_N, _H), jnp.float32),
            pltpu.VMEM((_N, _H), jnp.bfloat16),
            pltpu.VMEM((_N, _H), jnp.bfloat16),
            pltpu.VMEM((_N, _H), jnp.float32),
        ],
        compiler_params=pltpu.CompilerParams(
            dimension_semantics=("arbitrary", "arbitrary")),
    )(adj_bf, pro_gra, Wg0, bg0, wce, wg1p, bg1p)


def _ln(x, g, b):
    mu = jnp.mean(x, axis=-1, keepdims=True)
    var = jnp.mean((x - mu) * (x - mu), axis=-1, keepdims=True)
    return (x - mu) * jax.lax.rsqrt(var + 1e-5) * g + b


def _bf(x):
    return x.astype(jnp.bfloat16)


def _pep_body(pep_ref, wt0_ref, bt0_ref, wqkv_ref, bqkv_ref, wo_ref, bo_ref,
              w1_ref, b1_ref, w2_ref, b2_ref, g1_ref, be1_ref, g2_ref,
              be2_ref, wt1_ref, bt1_ref, out_ref, kbd, vbd):
    M = pep_ref.shape[0]
    x = jnp.maximum(
        jnp.dot(pep_ref[...], wt0_ref[...],
                preferred_element_type=jnp.float32) + bt0_ref[...], 0.0)
    nlayers = wqkv_ref.shape[0]
    nheads, dh = 4, 16
    CH = M // 2
    for l in range(nlayers):
        qkv = jnp.dot(_bf(x), wqkv_ref[l],
                      preferred_element_type=jnp.float32) + bqkv_ref[l]
        q = _bf(qkv[:, :_H] * (1.0 / math.sqrt(dh)))
        kbd[...] = jnp.zeros((_H, nheads * M), jnp.bfloat16)
        vbd[...] = jnp.zeros((nheads * M, 128), jnp.bfloat16)
        for h in range(nheads):
            kh = qkv[:, _H + dh * h:_H + dh * (h + 1)]
            kbd[dh * h:dh * (h + 1), M * h:M * (h + 1)] = _bf(kh).T
            vbd[M * h:M * (h + 1), dh * h:dh * (h + 1)] = _bf(
                qkv[:, 2 * _H + dh * h:2 * _H + dh * (h + 1)])
            vbd[M * h:M * (h + 1), _H + h:_H + h + 1] = jnp.ones(
                (M, 1), jnp.bfloat16)
        ochunks = []
        for c in range(0, M, CH):
            s = jnp.dot(q[c:c + CH], kbd[...],
                        preferred_element_type=jnp.float32)
            e = jnp.exp(s).astype(jnp.bfloat16)
            oext = jnp.dot(e, vbd[...], preferred_element_type=jnp.float32)
            heads = [oext[:, dh * h:dh * (h + 1)]
                     / oext[:, _H + h:_H + h + 1] for h in range(nheads)]
            ochunks.append(jnp.concatenate(heads, axis=1))
        o = jnp.concatenate(ochunks, axis=0)
        o = jnp.dot(o, wo_ref[l], preferred_element_type=jnp.float32) + bo_ref[l]
        x = _ln(x + o, g1_ref[l], be1_ref[l])
        ff = jnp.maximum(
            jnp.dot(x, w1_ref[l], preferred_element_type=jnp.float32)
            + b1_ref[l], 0.0)
        ff = jnp.dot(ff, w2_ref[l], preferred_element_type=jnp.float32) + b2_ref[l]
        x = _ln(x + ff, g2_ref[l], be2_ref[l])
    out_ref[...] = (jnp.dot(x, wt1_ref[...],
                            preferred_element_type=jnp.float32) + bt1_ref[...])


def _pep_branch(pep_p, wt0p, bt0, Wqkv, bqkv, Wo, bo, W1, b1, W2, b2,
                g1, be1, g2, be2, wt1p, bt1p):
    M = pep_p.shape[0]
    args = (pep_p, wt0p, bt0, Wqkv, bqkv, Wo, bo, W1, b1, W2, b2,
            g1, be1, g2, be2, wt1p, bt1p)
    in_specs = [pl.BlockSpec(a.shape, lambda i, nd=a.ndim: (0,) * nd)
                for a in args]
    return pl.pallas_call(
        _pep_body,
        grid=(1,),
        in_specs=in_specs,
        out_specs=pl.BlockSpec((M, 128), lambda i: (0, 0)),
        out_shape=jax.ShapeDtypeStruct((M, 128), jnp.float32),
        scratch_shapes=[
            pltpu.VMEM((_H, 4 * M), jnp.bfloat16),
            pltpu.VMEM((4 * M, 128), jnp.bfloat16),
        ],
        compiler_params=pltpu.CompilerParams(
            dimension_semantics=("arbitrary",),
            vmem_limit_bytes=100 * 1024 * 1024),
    )(*args)


def kernel(pro_gra, pro_adj, pep_tra, Wg0, bg0, Wc, Wg1, bg1, Wt0, bt0,
           Wt1, bt1, Wqkv, bqkv, Wo, bo, W1, b1, W2, b2,
           ln1g, ln1b, ln2g, ln2b):
    lamda = 0.5
    nl = Wc.shape[0]
    thetas = [min(1.0, math.log(lamda / (i + 1) + 1.0)) for i in range(nl)]
    eye = jnp.eye(_H, dtype=jnp.float32)
    wce = jnp.stack([t * Wc[i] + (1.0 - t) * eye
                     for i, t in enumerate(thetas)])

    adj_bf = pro_adj.astype(jnp.bfloat16)
    wg1p = jnp.pad(Wg1, ((0, 0), (0, 128 - Wg1.shape[1])))
    bg1p = jnp.pad(bg1, (0, 128 - bg1.shape[0])).reshape(1, 128)
    gra_full = _graph_branch(adj_bf, pro_gra, Wg0, bg0.reshape(1, _H),
                             wce, wg1p, bg1p)

    M = pep_tra.shape[0]
    L = Wqkv.shape[0]
    pep_p = jnp.concatenate(
        [pep_tra[:, :50], pep_tra[:, 62:],
         jnp.zeros((M, 12), jnp.float32)], axis=1)
    wt0p = jnp.concatenate(
        [Wt0, jnp.zeros((12, Wt0.shape[1]), jnp.float32)], axis=0)
    wt1p = jnp.pad(Wt1, ((0, 0), (0, 128 - Wt1.shape[1])))
    bt1p = jnp.pad(bt1, (0, 128 - bt1.shape[0])).reshape(1, 128)
    pep_full = _pep_branch(
        pep_p, wt0p, bt0.reshape(1, _H),
        Wqkv.astype(jnp.bfloat16), bqkv.reshape(L, 1, 3 * _H),
        Wo, bo.reshape(L, 1, _H),
        W1, b1.reshape(L, 1, 4 * _H), W2, b2.reshape(L, 1, _H),
        ln1g.reshape(L, 1, _H), ln1b.reshape(L, 1, _H),
        ln2g.reshape(L, 1, _H), ln2b.reshape(L, 1, _H), wt1p, bt1p)

    nc = Wg1.shape[1]
    return jnp.concatenate([gra_full[:, :nc], pep_full[:, :nc]], axis=0)


# speedup vs baseline: 1.0558x; 1.0558x over previous
import math

import jax
import jax.numpy as jnp
from jax.experimental import pallas as pl
from jax.experimental.pallas import tpu as pltpu

_N = 4096
_F = 128
_H = 64
_NL = 8
_BR = 2048
_NB = _N // _BR
_ALPHA = 0.1
_M = 2048
_DH = 16
_NH = 4
_CB = 256
_NCB = _N // _CB
_CH = 512


def _ln(x, g, b):
    mu = jnp.mean(x, axis=-1, keepdims=True)
    var = jnp.mean((x - mu) * (x - mu), axis=-1, keepdims=True)
    return (x - mu) * jax.lax.rsqrt(var + 1e-5) * g + b


def _bf(x):
    return x.astype(jnp.bfloat16)


def _castpep_body(adj_ref, pep_ref, wt0_ref, bt0_ref, wqkv_ref, bqkv_ref,
                  wo_ref, bo_ref, w1_ref, b1_ref, w2_ref, b2_ref,
                  g1_ref, be1_ref, g2_ref, be2_ref, wt1_ref, bt1_ref,
                  adjbf_ref, pout_ref,
                  xsc, qkvsc, kbd, vbd, osc):
    step = pl.program_id(0)
    adjbf_ref[...] = adj_ref[...].astype(jnp.bfloat16)

    def qkv_phase(l):
        qkv = (jnp.dot(_bf(xsc[...]), wqkv_ref[l],
                       preferred_element_type=jnp.float32) + bqkv_ref[l])
        qkvsc[...] = qkv
        kbd[...] = jnp.zeros((_H, _NH * _M), jnp.bfloat16)
        vbd[...] = jnp.zeros((_NH * _M, 128), jnp.bfloat16)
        for h in range(_NH):
            kh = qkv[:, _H + _DH * h:_H + _DH * (h + 1)]
            kbd[_DH * h:_DH * (h + 1), _M * h:_M * (h + 1)] = _bf(kh).T
            vbd[_M * h:_M * (h + 1), _DH * h:_DH * (h + 1)] = _bf(
                qkv[:, 2 * _H + _DH * h:2 * _H + _DH * (h + 1)])
            vbd[_M * h:_M * (h + 1), _H + h:_H + h + 1] = jnp.ones(
                (_M, 1), jnp.bfloat16)

    def projff_phase(l):
        o = (jnp.dot(osc[...], wo_ref[l],
                     preferred_element_type=jnp.float32) + bo_ref[l])
        x = _ln(xsc[...] + o, g1_ref[l], be1_ref[l])
        ff = jnp.maximum(
            jnp.dot(x, w1_ref[l], preferred_element_type=jnp.float32)
            + b1_ref[l], 0.0)
        ff = (jnp.dot(ff, w2_ref[l], preferred_element_type=jnp.float32)
              + b2_ref[l])
        xsc[...] = _ln(x + ff, g2_ref[l], be2_ref[l])

    @pl.when(step == 0)
    def _embed_qkv0():
        xsc[...] = jnp.maximum(
            jnp.dot(pep_ref[...], wt0_ref[...],
                    preferred_element_type=jnp.float32) + bt0_ref[...], 0.0)
        qkv_phase(0)

    is_attn = jnp.logical_or(
        jnp.logical_and(step >= 1, step <= 4),
        jnp.logical_and(step >= 7, step <= 10))

    @pl.when(is_attn)
    def _attn_chunk():
        c = (step - jnp.where(step >= 7, 7, 1)) * _CH
        q = _bf(qkvsc[pl.ds(c, _CH), :_H] * (1.0 / math.sqrt(_DH)))
        s = jnp.dot(q, kbd[...], preferred_element_type=jnp.float32)
        e = jnp.exp(s).astype(jnp.bfloat16)
        oext = jnp.dot(e, vbd[...], preferred_element_type=jnp.float32)
        heads = [oext[:, _DH * h:_DH * (h + 1)]
                 / oext[:, _H + h:_H + h + 1] for h in range(_NH)]
        osc[pl.ds(c, _CH), :] = jnp.concatenate(heads, axis=1)

    @pl.when(step == 5)
    def _projff0():
        projff_phase(0)

    @pl.when(step == 6)
    def _qkv1():
        qkv_phase(1)

    @pl.when(step == 11)
    def _projff1():
        projff_phase(1)

    @pl.when(step == 12)
    def _final():
        pout_ref[...] = (jnp.dot(xsc[...], wt1_ref[...],
                                 preferred_element_type=jnp.float32)
                         + bt1_ref[...])


def _castpep(pro_adj, pep_p, wt0p, bt0, Wqkv, bqkv, Wo, bo, W1, b1, W2, b2,
             g1, be1, g2, be2, wt1p, bt1p):
    args = (pro_adj, pep_p, wt0p, bt0, Wqkv, bqkv, Wo, bo, W1, b1, W2, b2,
            g1, be1, g2, be2, wt1p, bt1p)
    in_specs = [pl.BlockSpec((_CB, _N), lambda c: (c, 0))]
    in_specs += [pl.BlockSpec(a.shape, lambda c, nd=a.ndim: (0,) * nd)
                 for a in args[1:]]
    return pl.pallas_call(
        _castpep_body,
        grid=(_NCB,),
        in_specs=in_specs,
        out_specs=[
            pl.BlockSpec((_CB, _N), lambda c: (c, 0)),
            pl.BlockSpec((_M, 128), lambda c: (0, 0)),
        ],
        out_shape=[
            jax.ShapeDtypeStruct((_N, _N), jnp.bfloat16),
            jax.ShapeDtypeStruct((_M, 128), jnp.float32),
        ],
        scratch_shapes=[
            pltpu.VMEM((_M, _H), jnp.float32),
            pltpu.VMEM((_M, 3 * _H), jnp.float32),
            pltpu.VMEM((_H, _NH * _M), jnp.bfloat16),
            pltpu.VMEM((_NH * _M, 128), jnp.bfloat16),
            pltpu.VMEM((_M, _H), jnp.float32),
        ],
        compiler_params=pltpu.CompilerParams(
            dimension_semantics=("arbitrary",),
            vmem_limit_bytes=100 * 1024 * 1024),
    )(*args)


def _gcn_body(adj_ref, gra_ref, wg0_ref, bg0_ref, wce_ref, wg1_ref, bg1_ref,
              out_ref, xf0, xf1, xb0, xb1, h0s):
    i = pl.program_id(0)
    r = pl.program_id(1)
    row0 = r * _BR

    @pl.when(jnp.logical_and(i == 0, r == 0))
    def _prologue():
        x0 = jnp.maximum(
            jnp.dot(gra_ref[...], wg0_ref[...],
                    preferred_element_type=jnp.float32) + bg0_ref[...], 0.0)
        xf0[...] = x0
        xb0[...] = ((1.0 - _ALPHA) * x0).astype(jnp.bfloat16)
        h0s[...] = _ALPHA * x0

    def step(src_f, src_b, dst_f, dst_b):
        hi = jnp.dot(adj_ref[...], src_b[...],
                     preferred_element_type=jnp.float32)
        support = hi + h0s[pl.ds(row0, _BR), :]
        out = jnp.dot(support, wce_ref[0],
                      preferred_element_type=jnp.float32)
        xn = jnp.maximum(out + src_f[pl.ds(row0, _BR), :], 0.0)
        dst_f[pl.ds(row0, _BR), :] = xn
        dst_b[pl.ds(row0, _BR), :] = ((1.0 - _ALPHA) * xn).astype(jnp.bfloat16)

        @pl.when(i == _NL - 1)
        def _epilogue():
            out_ref[...] = (jnp.dot(xn, wg1_ref[...],
                                    preferred_element_type=jnp.float32)
                            + bg1_ref[...])

    @pl.when(i % 2 == 0)
    def _even():
        step(xf0, xb0, xf1, xb1)

    @pl.when(i % 2 == 1)
    def _odd():
        step(xf1, xb1, xf0, xb0)


def _graph_branch(adj_bf, pro_gra, Wg0, bg0, wce, wg1p, bg1p):
    return pl.pallas_call(
        _gcn_body,
        grid=(_NL, _NB),
        in_specs=[
            pl.BlockSpec((_BR, _N), lambda i, r: (r, 0)),
            pl.BlockSpec((_N, _F), lambda i, r: (0, 0)),
            pl.BlockSpec((_F, _H), lambda i, r: (0, 0)),
            pl.BlockSpec((1, _H), lambda i, r: (0, 0)),
            pl.BlockSpec((1, _H, _H), lambda i, r: (i, 0, 0)),
            pl.BlockSpec((_H, 128), lambda i, r: (0, 0)),
            pl.BlockSpec((1, 128), lambda i, r: (0, 0)),
        ],
        out_specs=pl.BlockSpec((_BR, 128), lambda i, r: (r, 0)),
        out_shape=jax.ShapeDtypeStruct((_N, 128), jnp.float32),
        scratch_shapes=[
            pltpu.VMEM((_N, _H), jnp.float32),
            pltpu.VMEM((_N, _H), jnp.float32),
            pltpu.VMEM((_N, _H), jnp.bfloat16),
            pltpu.VMEM((_N, _H), jnp.bfloat16),
            pltpu.VMEM((_N, _H), jnp.float32),
        ],
        compiler_params=pltpu.CompilerParams(
            dimension_semantics=("arbitrary", "arbitrary")),
    )(adj_bf, pro_gra, Wg0, bg0, wce, wg1p, bg1p)


def kernel(pro_gra, pro_adj, pep_tra, Wg0, bg0, Wc, Wg1, bg1, Wt0, bt0,
           Wt1, bt1, Wqkv, bqkv, Wo, bo, W1, b1, W2, b2,
           ln1g, ln1b, ln2g, ln2b):
    lamda = 0.5
    nl = Wc.shape[0]
    thetas = [min(1.0, math.log(lamda / (i + 1) + 1.0)) for i in range(nl)]
    eye = jnp.eye(_H, dtype=jnp.float32)
    wce = jnp.stack([t * Wc[i] + (1.0 - t) * eye
                     for i, t in enumerate(thetas)])

    M = pep_tra.shape[0]
    L = Wqkv.shape[0]
    pep_p = jnp.concatenate(
        [pep_tra[:, :50], pep_tra[:, 62:],
         jnp.zeros((M, 12), jnp.float32)], axis=1)
    wt0p = jnp.concatenate(
        [Wt0, jnp.zeros((12, Wt0.shape[1]), jnp.float32)], axis=0)
    wt1p = jnp.pad(Wt1, ((0, 0), (0, 128 - Wt1.shape[1])))
    bt1p = jnp.pad(bt1, (0, 128 - bt1.shape[0])).reshape(1, 128)

    adj_bf, pep_full = _castpep(
        pro_adj, pep_p, wt0p, bt0.reshape(1, _H),
        Wqkv.astype(jnp.bfloat16), bqkv.reshape(L, 1, 3 * _H),
        Wo, bo.reshape(L, 1, _H),
        W1, b1.reshape(L, 1, 4 * _H), W2, b2.reshape(L, 1, _H),
        ln1g.reshape(L, 1, _H), ln1b.reshape(L, 1, _H),
        ln2g.reshape(L, 1, _H), ln2b.reshape(L, 1, _H), wt1p, bt1p)

    wg1p = jnp.pad(Wg1, ((0, 0), (0, 128 - Wg1.shape[1])))
    bg1p = jnp.pad(bg1, (0, 128 - bg1.shape[0])).reshape(1, 128)
    gra_full = _graph_branch(adj_bf, pro_gra, Wg0, bg0.reshape(1, _H),
                             wce, wg1p, bg1p)

    nc = Wg1.shape[1]
    return jnp.concatenate([gra_full[:, :nc], pep_full[:, :nc]], axis=0)
